# baseline (speedup 1.0000x reference)
import jax
import jax.numpy as jnp
from jax import lax
from jax.experimental import pallas as pl
from jax.experimental.pallas import tpu as pltpu
from jax.experimental.pallas import tpu_sc as plsc

D = 1024
S = 2048
B = 4
NW = 32
PPW = S // NW
C = 8
NPC = PPW // C
NGRP = 3
LANES = 16
KB = 8


def _emb_body(ids_hbm, wte_hbm, wpe_hbm, out_hbm,
              idx_v, rowsb, wpeb, gsem, ssem, wsem):
    wid = lax.axis_index("s") * 2 + lax.axis_index("c")
    pos_base = wid * PPW

    for b in range(B):
        pltpu.sync_copy(ids_hbm.at[b, pl.ds(pos_base, PPW)], idx_v.at[b])

    def wpe_copy(g):
        return pltpu.make_async_copy(
            wpe_hbm.at[pl.ds(pos_base + g * C, C)], wpeb.at[g % 2],
            wsem.at[g % 2])

    def gather_copy(g, b):
        return pltpu.make_async_copy(
            wte_hbm.at[idx_v.at[b, pl.ds(g * C, C)]], rowsb.at[g % NGRP, b],
            gsem.at[g % NGRP, b])

    def store_copy(g, b):
        return pltpu.make_async_copy(
            rowsb.at[g % NGRP, b],
            out_hbm.at[pl.ds(b * S + pos_base + g * C, C)],
            ssem.at[g % NGRP, b])

    wpe_copy(0).start()
    wpe_copy(1).start()
    for g in range(2):
        for b in range(B):
            gather_copy(g, b).start()

    def group_body(g, carry):
        p = g % NGRP
        wb = g % 2
        wpe_copy(g).wait()

        def batch_body(b, c):
            gather_copy(g, b).wait()

            def add_row(r, c1):
                for k in range(D // LANES):
                    sl = pl.ds(k * LANES, LANES)
                    plsc.addupdate(rowsb.at[p, b, r, sl], wpeb[wb, r, sl])
                return c1

            lax.fori_loop(0, C, add_row, 0)
            store_copy(g, b).start()
            return c

        lax.fori_loop(0, B, batch_body, 0)

        def prefetch(_):
            wpe_copy(g + 2).start()

            def next_gather(b, c):
                def drain(_):
                    store_copy(g - 1, b).wait()
                    return 0

                lax.cond(g >= 1, drain, lambda _: 0, 0)
                gather_copy(g + 2, b).start()
                return c

            lax.fori_loop(0, B, next_gather, 0)
            return 0

        lax.cond(g + 2 < NPC, prefetch, lambda _: 0, 0)
        return carry

    lax.fori_loop(0, NPC, group_body, 0)

    def drain_tail(g, c):
        def drain_b(b, c2):
            store_copy(g, b).wait()
            return c2

        lax.fori_loop(0, B, drain_b, 0)
        return c

    lax.fori_loop(NPC - NGRP, NPC, drain_tail, 0)


def kernel(input_ids, wte, wpe):
    ids = input_ids.astype(jnp.int32)
    mesh = plsc.VectorSubcoreMesh(core_axis_name="c", subcore_axis_name="s")
    f = pl.kernel(
        _emb_body,
        out_type=jax.ShapeDtypeStruct((B * S, D), jnp.float32),
        mesh=mesh,
        scratch_types=(
            pltpu.VMEM((B, PPW), jnp.int32),
            pltpu.VMEM((NGRP, B, C, D), jnp.float32),
            pltpu.VMEM((2, C, D), jnp.float32),
            pltpu.SemaphoreType.DMA((NGRP, B)),
            pltpu.SemaphoreType.DMA((NGRP, B)),
            pltpu.SemaphoreType.DMA((2,)),
        ),
    )
    out = f(ids, wte, wpe)
    return out.reshape(B, S, D)


# speedup vs baseline: 1.6606x; 1.6606x over previous
import jax
import jax.numpy as jnp
from jax import lax
from jax.experimental import pallas as pl
from jax.experimental.pallas import tpu as pltpu
from jax.experimental.pallas import tpu_sc as plsc

D = 1024
S = 2048
B = 4
NW = 32
PPW = S // NW
C = 8
NPC = PPW // C
NGRP = 3
LANES = 16
KB = 8


def _emb_body(ids_hbm, wte_hbm, wpe_hbm, out_hbm,
              idx_v, rowsb, wpeb, gsem, ssem, wsem):
    wid = lax.axis_index("s") * 2 + lax.axis_index("c")
    pos_base = wid * PPW

    for b in range(B):
        pltpu.sync_copy(ids_hbm.at[b, pl.ds(pos_base, PPW)], idx_v.at[b])

    def wpe_copy(g):
        return pltpu.make_async_copy(
            wpe_hbm.at[pl.ds(pos_base + g * C, C)], wpeb.at[g % 2],
            wsem.at[g % 2])

    def gather_copy(g, b):
        return pltpu.make_async_copy(
            wte_hbm.at[idx_v.at[b, pl.ds(g * C, C)]], rowsb.at[g % NGRP, b],
            gsem.at[g % NGRP, b])

    def store_copy(g, b):
        return pltpu.make_async_copy(
            rowsb.at[g % NGRP, b],
            out_hbm.at[pl.ds(b * S + pos_base + g * C, C)],
            ssem.at[g % NGRP, b])

    wpe_copy(0).start()
    wpe_copy(1).start()
    for g in range(2):
        for b in range(B):
            gather_copy(g, b).start()

    def group_body(g, carry):
        p = g % NGRP
        wb = g % 2
        wpe_copy(g).wait()

        def wait_g(b, c):
            gather_copy(g, b).wait()
            return c

        lax.fori_loop(0, B, wait_g, 0)

        def add_row(r, c1):
            for k in range(D // LANES):
                sl = pl.ds(k * LANES, LANES)
                v = wpeb[wb, r, sl]
                for b in range(B):
                    plsc.addupdate(rowsb.at[p, b, r, sl], v)
            return c1

        lax.fori_loop(0, C, add_row, 0)

        def issue_store(b, c):
            store_copy(g, b).start()
            return c

        lax.fori_loop(0, B, issue_store, 0)

        def prefetch(_):
            wpe_copy(g + 2).start()

            def next_gather(b, c):
                def drain(_):
                    store_copy(g - 1, b).wait()
                    return 0

                lax.cond(g >= 1, drain, lambda _: 0, 0)
                gather_copy(g + 2, b).start()
                return c

            lax.fori_loop(0, B, next_gather, 0)
            return 0

        lax.cond(g + 2 < NPC, prefetch, lambda _: 0, 0)
        return carry

    lax.fori_loop(0, NPC, group_body, 0)

    def drain_tail(g, c):
        def drain_b(b, c2):
            store_copy(g, b).wait()
            return c2

        lax.fori_loop(0, B, drain_b, 0)
        return c

    lax.fori_loop(NPC - NGRP, NPC, drain_tail, 0)


def kernel(input_ids, wte, wpe):
    ids = input_ids.astype(jnp.int32)
    mesh = plsc.VectorSubcoreMesh(core_axis_name="c", subcore_axis_name="s")
    f = pl.kernel(
        _emb_body,
        out_type=jax.ShapeDtypeStruct((B * S, D), jnp.float32),
        mesh=mesh,
        scratch_types=(
            pltpu.VMEM((B, PPW), jnp.int32),
            pltpu.VMEM((NGRP, B, C, D), jnp.float32),
            pltpu.VMEM((2, C, D), jnp.float32),
            pltpu.SemaphoreType.DMA((NGRP, B)),
            pltpu.SemaphoreType.DMA((NGRP, B)),
            pltpu.SemaphoreType.DMA((2,)),
        ),
    )
    out = f(ids, wte, wpe)
    return out.reshape(B, S, D)
